# initial kernel scaffold (unmeasured)
import jax
import jax.numpy as jnp
from jax import lax
from jax.experimental import pallas as pl
from jax.experimental.pallas import tpu as pltpu

NZ = 4
FC = 512


def kernel(x, assign, W1, W2):
    T, D = x.shape
    EL, _, F = W1.shape
    nfc = F // FC

    xb = x.astype(jnp.bfloat16)
    w1b = W1.astype(jnp.bfloat16)
    w2b = W2.astype(jnp.bfloat16)
    a2 = assign.reshape(T, 1)

    def body(xb_ref, a_ref, w1_ref, w2_ref, out_ref,
             comm, agc, w1e, w2e, acc, rbuf,
             ld, xs, xr, gs, gr, rs, rr):
        p = lax.axis_index("z")
        mx = lax.axis_index("x")
        my = lax.axis_index("y")
        right = lax.rem(p + 1, NZ)
        left = lax.rem(p + NZ - 1, NZ)

        cp_x = pltpu.make_async_copy(xb_ref, comm.at[0], ld.at[0])
        cp_x.start()
        cp_a = pltpu.make_async_copy(a_ref, agc.at[0], ld.at[1])
        cp_a.start()

        barrier = pltpu.get_barrier_semaphore()
        pl.semaphore_signal(barrier, inc=1, device_id=(mx, my, left),
                            device_id_type=pl.DeviceIdType.MESH)
        pl.semaphore_signal(barrier, inc=1, device_id=(mx, my, right),
                            device_id_type=pl.DeviceIdType.MESH)
        pl.semaphore_wait(barrier, 2)

        cp_x.wait()
        cp_a.wait()

        def compute(slot, tgt, tdt):
            tgt[...] = jnp.zeros((T, D), tdt)
            agv = agc[slot]

            def e_body(e, _):
                gid = p * EL + e

                def f_body(fc, _):
                    l1 = pltpu.make_async_copy(
                        w1_ref.at[e, :, pl.ds(fc * FC, FC)], w1e, ld.at[0])
                    l1.start()
                    l2 = pltpu.make_async_copy(
                        w2_ref.at[e, pl.ds(fc * FC, FC), :], w2e, ld.at[1])
                    l2.start()
                    l1.wait()
                    l2.wait()
                    h = jnp.dot(comm[slot], w1e[...],
                                preferred_element_type=jnp.float32)
                    hb = jnp.maximum(h, 0.0).astype(jnp.bfloat16)
                    contr = jnp.dot(hb, w2e[...],
                                    preferred_element_type=jnp.float32)
                    masked = jnp.where(agv == gid, contr, 0.0)
                    tgt[...] = (tgt[...].astype(jnp.float32)
                                + masked).astype(tdt)
                    return 0

                lax.fori_loop(0, nfc, f_body, 0)
                return 0

            lax.fori_loop(0, EL, e_body, 0)

        def ring(h):
            rx = pltpu.make_async_remote_copy(
                src_ref=comm.at[h], dst_ref=comm.at[h + 1],
                send_sem=xs.at[h], recv_sem=xr.at[h],
                device_id=(mx, my, right),
                device_id_type=pl.DeviceIdType.MESH)
            rg = pltpu.make_async_remote_copy(
                src_ref=agc.at[h], dst_ref=agc.at[h + 1],
                send_sem=gs.at[h], recv_sem=gr.at[h],
                device_id=(mx, my, right),
                device_id_type=pl.DeviceIdType.MESH)
            return rx, rg

        def ret(h, a_slot):
            o = lax.rem(p + NZ - 1 - h, NZ)
            return pltpu.make_async_remote_copy(
                src_ref=acc.at[a_slot], dst_ref=rbuf.at[h],
                send_sem=rs.at[h], recv_sem=rr.at[h],
                device_id=(mx, my, o),
                device_id_type=pl.DeviceIdType.MESH)

        rx0, rg0 = ring(0)
        rx0.start()
        rg0.start()
        compute(0, out_ref, jnp.float32)
        rx0.wait()
        rg0.wait()

        rx1, rg1 = ring(1)
        rx1.start()
        rg1.start()
        compute(1, acc.at[0], jnp.bfloat16)
        r0 = ret(0, 0)
        r0.start()
        rx1.wait()
        rg1.wait()

        rx2, rg2 = ring(2)
        rx2.start()
        rg2.start()
        compute(2, acc.at[1], jnp.bfloat16)
        r1 = ret(1, 1)
        r1.start()
        rx2.wait()
        rg2.wait()

        r0.wait_send()
        compute(3, acc.at[0], jnp.bfloat16)
        r2 = ret(2, 0)
        r2.start()
        r1.wait_send()
        r2.wait_send()

        for h in range(NZ - 1):
            ret(h, 0).wait_recv()
        out_ref[...] = (out_ref[...]
                        + rbuf[0].astype(jnp.float32)
                        + rbuf[1].astype(jnp.float32)
                        + rbuf[2].astype(jnp.float32))

    try:
        cparams = pltpu.CompilerParams(collective_id=0)
    except AttributeError:
        cparams = pltpu.TPUCompilerParams(collective_id=0)

    return pl.pallas_call(
        body,
        out_shape=jax.ShapeDtypeStruct((T, D), jnp.float32),
        in_specs=[
            pl.BlockSpec(memory_space=pltpu.ANY),
            pl.BlockSpec(memory_space=pltpu.ANY),
            pl.BlockSpec(memory_space=pltpu.ANY),
            pl.BlockSpec(memory_space=pltpu.ANY),
        ],
        out_specs=pl.BlockSpec(memory_space=pltpu.VMEM),
        scratch_shapes=[
            pltpu.VMEM((NZ, T, D), jnp.bfloat16),
            pltpu.VMEM((NZ, T, 1), jnp.int32),
            pltpu.VMEM((D, FC), jnp.bfloat16),
            pltpu.VMEM((FC, D), jnp.bfloat16),
            pltpu.VMEM((2, T, D), jnp.bfloat16),
            pltpu.VMEM((NZ - 1, T, D), jnp.bfloat16),
            pltpu.SemaphoreType.DMA((2,)),
            pltpu.SemaphoreType.DMA((NZ - 1,)),
            pltpu.SemaphoreType.DMA((NZ - 1,)),
            pltpu.SemaphoreType.DMA((NZ - 1,)),
            pltpu.SemaphoreType.DMA((NZ - 1,)),
            pltpu.SemaphoreType.DMA((NZ - 1,)),
            pltpu.SemaphoreType.DMA((NZ - 1,)),
        ],
        compiler_params=cparams,
    )(xb, a2, w1b, w2b)


# baseline (device time: 521242 ns/iter reference)
import jax
import jax.numpy as jnp
from jax import lax
from jax.experimental import pallas as pl
from jax.experimental.pallas import tpu as pltpu

NZ = 4
FC = 512


def kernel(x, assign, W1, W2):
    T, D = x.shape
    EL, _, F = W1.shape
    nfc = F // FC

    xb = x.astype(jnp.bfloat16)
    w1b = W1.astype(jnp.bfloat16)
    w2b = W2.astype(jnp.bfloat16)
    a2 = assign.reshape(T, 1)

    def body(xb_ref, a_ref, w1_ref, w2_ref, out_ref,
             comm, agc, w1e, w2e, acc, rbuf,
             ld, xs, xr, gs, gr, rs, rr):
        p = lax.axis_index("z")
        mx = lax.axis_index("x")
        my = lax.axis_index("y")
        right = lax.rem(p + 1, NZ)
        left = lax.rem(p + NZ - 1, NZ)

        cp_x = pltpu.make_async_copy(xb_ref, comm.at[0], ld.at[0])
        cp_x.start()
        cp_a = pltpu.make_async_copy(a_ref, agc.at[0], ld.at[1])
        cp_a.start()

        barrier = pltpu.get_barrier_semaphore()
        pl.semaphore_signal(barrier, inc=1, device_id=(mx, my, left),
                            device_id_type=pl.DeviceIdType.MESH)
        pl.semaphore_signal(barrier, inc=1, device_id=(mx, my, right),
                            device_id_type=pl.DeviceIdType.MESH)
        pl.semaphore_wait(barrier, 2)

        cp_x.wait()
        cp_a.wait()

        def compute(slot, tgt, tdt):
            tgt[...] = jnp.zeros((T, D), tdt)
            agv = agc[slot]

            def e_body(e, _):
                gid = p * EL + e

                def f_body(fc, _):
                    l1 = pltpu.make_async_copy(
                        w1_ref.at[e, :, pl.ds(fc * FC, FC)], w1e, ld.at[0])
                    l1.start()
                    l2 = pltpu.make_async_copy(
                        w2_ref.at[e, pl.ds(fc * FC, FC), :], w2e, ld.at[1])
                    l2.start()
                    l1.wait()
                    l2.wait()
                    h = jnp.dot(comm[slot], w1e[...],
                                preferred_element_type=jnp.float32)
                    hb = jnp.maximum(h, 0.0).astype(jnp.bfloat16)
                    contr = jnp.dot(hb, w2e[...],
                                    preferred_element_type=jnp.float32)
                    masked = jnp.where(agv == gid, contr, 0.0)
                    tgt[...] = (tgt[...].astype(jnp.float32)
                                + masked).astype(tdt)
                    return 0

                lax.fori_loop(0, nfc, f_body, 0)
                return 0

            lax.fori_loop(0, EL, e_body, 0)

        def ring(h):
            rx = pltpu.make_async_remote_copy(
                src_ref=comm.at[h], dst_ref=comm.at[h + 1],
                send_sem=xs.at[h], recv_sem=xr.at[h],
                device_id=(mx, my, right),
                device_id_type=pl.DeviceIdType.MESH)
            rg = pltpu.make_async_remote_copy(
                src_ref=agc.at[h], dst_ref=agc.at[h + 1],
                send_sem=gs.at[h], recv_sem=gr.at[h],
                device_id=(mx, my, right),
                device_id_type=pl.DeviceIdType.MESH)
            return rx, rg

        def ret(h, a_slot):
            o = lax.rem(p + NZ - 1 - h, NZ)
            return pltpu.make_async_remote_copy(
                src_ref=acc.at[a_slot], dst_ref=rbuf.at[h],
                send_sem=rs.at[h], recv_sem=rr.at[h],
                device_id=(mx, my, o),
                device_id_type=pl.DeviceIdType.MESH)

        rx0, rg0 = ring(0)
        rx0.start()
        rg0.start()
        compute(0, out_ref, jnp.float32)
        rx0.wait()
        rg0.wait()

        rx1, rg1 = ring(1)
        rx1.start()
        rg1.start()
        compute(1, acc.at[0], jnp.bfloat16)
        r0 = ret(0, 0)
        r0.start()
        rx1.wait()
        rg1.wait()

        rx2, rg2 = ring(2)
        rx2.start()
        rg2.start()
        compute(2, acc.at[1], jnp.bfloat16)
        r1 = ret(1, 1)
        r1.start()
        rx2.wait()
        rg2.wait()

        r0.wait_send()
        compute(3, acc.at[0], jnp.bfloat16)
        r2 = ret(2, 0)
        r2.start()
        r1.wait_send()
        r2.wait_send()

        for h in range(NZ - 1):
            ret(h, 0).wait_recv()
        out_ref[...] = (out_ref[...]
                        + rbuf[0].astype(jnp.float32)
                        + rbuf[1].astype(jnp.float32)
                        + rbuf[2].astype(jnp.float32))

    cparams = pltpu.CompilerParams(
        collective_id=0, vmem_limit_bytes=100 * 1024 * 1024)

    return pl.pallas_call(
        body,
        out_shape=jax.ShapeDtypeStruct((T, D), jnp.float32),
        in_specs=[
            pl.BlockSpec(memory_space=pl.ANY),
            pl.BlockSpec(memory_space=pl.ANY),
            pl.BlockSpec(memory_space=pl.ANY),
            pl.BlockSpec(memory_space=pl.ANY),
        ],
        out_specs=pl.BlockSpec(memory_space=pltpu.VMEM),
        scratch_shapes=[
            pltpu.VMEM((NZ, T, D), jnp.bfloat16),
            pltpu.VMEM((NZ, T, 1), jnp.int32),
            pltpu.VMEM((D, FC), jnp.bfloat16),
            pltpu.VMEM((FC, D), jnp.bfloat16),
            pltpu.VMEM((2, T, D), jnp.bfloat16),
            pltpu.VMEM((NZ - 1, T, D), jnp.bfloat16),
            pltpu.SemaphoreType.DMA((2,)),
            pltpu.SemaphoreType.DMA((NZ - 1,)),
            pltpu.SemaphoreType.DMA((NZ - 1,)),
            pltpu.SemaphoreType.DMA((NZ - 1,)),
            pltpu.SemaphoreType.DMA((NZ - 1,)),
            pltpu.SemaphoreType.DMA((NZ - 1,)),
            pltpu.SemaphoreType.DMA((NZ - 1,)),
        ],
        compiler_params=cparams,
    )(xb, a2, w1b, w2b)
